# Initial kernel scaffold; baseline (speedup 1.0000x reference)
import jax
import jax.numpy as jnp
from jax.experimental import pallas as pl


def kernel(gt_centerness, pred_binary):
    raise NotImplementedError("write your pallas kernel here")


# speedup vs baseline: 16.3893x; 16.3893x over previous
import functools

import jax
import jax.numpy as jnp
from jax import lax
from jax.experimental import pallas as pl
from jax.experimental.pallas import tpu as pltpu
from jax.experimental.pallas import tpu_sc as plsc

NEG_RATIO = 3.0
EPS = 1e-4
WEIGHT = 1.0

N_TOTAL = 16 * 512 * 512
ROWS = 2048
COLS = 2048
BLK_ROWS = 128
GRID = ROWS // BLK_ROWS

NUM_WORKERS = 32
PER_TILE = N_TOTAL // NUM_WORKERS
CHUNK = 16384
N_CHUNKS = PER_TILE // CHUNK
NB = 8576
SHIFT = 17


def _bce_body(gt_ref, pred_ref, neg_ref, stats_ref):
    g = gt_ref[...]
    p = pred_ref[...]
    logp = jnp.maximum(jnp.log(p), -100.0)
    log1mp = jnp.maximum(jnp.log(1.0 - p), -100.0)
    loss = -(g * logp + (1.0 - g) * log1mp)
    pos_m = g >= 0.9
    ign_m = jnp.logical_and(g >= 0.8, g < 0.9)
    neg_m = g < 0.8
    neg_ref[...] = jnp.where(neg_m, loss, 0.0)
    pos_cnt = jnp.sum(pos_m.astype(jnp.float32))
    ign_cnt = jnp.sum(ign_m.astype(jnp.float32))
    pos_sum = jnp.sum(jnp.where(pos_m, loss, 0.0))
    lane = lax.broadcasted_iota(jnp.int32, (1, 1, 128), 2)
    stats_ref[...] = jnp.where(
        lane == 0, pos_cnt,
        jnp.where(lane == 1, ign_cnt,
                  jnp.where(lane == 2, pos_sum, 0.0)))


def _tc_bce(gt2d, pred2d):
    return pl.pallas_call(
        _bce_body,
        grid=(GRID,),
        in_specs=[
            pl.BlockSpec((BLK_ROWS, COLS), lambda i: (i, 0)),
            pl.BlockSpec((BLK_ROWS, COLS), lambda i: (i, 0)),
        ],
        out_specs=[
            pl.BlockSpec((BLK_ROWS, COLS), lambda i: (i, 0)),
            pl.BlockSpec((1, 1, 128), lambda i: (i, 0, 0)),
        ],
        out_shape=[
            jax.ShapeDtypeStruct((ROWS, COLS), jnp.float32),
            jax.ShapeDtypeStruct((GRID, 1, 128), jnp.float32),
        ],
    )(gt2d, pred2d)


def _sc_hist_kernel(neg_hbm, out_hbm, buf, cnt_ref, sum_ref):
    c = lax.axis_index("c")
    s = lax.axis_index("s")
    wid = s * 2 + c
    base = wid * PER_TILE

    zeros16 = jnp.zeros((16,), jnp.float32)
    ones16 = jnp.ones((16,), jnp.float32)

    def zero_body(i, carry):
        cnt_ref[pl.ds(i * 16, 16)] = zeros16
        sum_ref[pl.ds(i * 16, 16)] = zeros16
        return carry

    lax.fori_loop(0, NB // 16, zero_body, 0)

    def chunk_body(ci, carry):
        pltpu.sync_copy(neg_hbm.at[pl.ds(base + ci * CHUNK, CHUNK)], buf)

        def inner(j, c2):
            x = buf[pl.ds(j * 16, 16)]
            bits = lax.bitcast_convert_type(x, jnp.int32)
            b = jnp.minimum(lax.shift_right_logical(bits, SHIFT), NB - 1)
            plsc.addupdate_scatter(cnt_ref, [b], ones16)
            plsc.addupdate_scatter(sum_ref, [b], x)
            return c2

        lax.fori_loop(0, CHUNK // 16, inner, 0)
        return carry

    lax.fori_loop(0, N_CHUNKS, chunk_body, 0)

    pltpu.sync_copy(cnt_ref, out_hbm.at[2 * wid])
    pltpu.sync_copy(sum_ref, out_hbm.at[2 * wid + 1])


def _sc_hist(neg_flat):
    mesh = plsc.VectorSubcoreMesh(core_axis_name="c", subcore_axis_name="s")
    fn = functools.partial(
        pl.kernel,
        mesh=mesh,
        out_type=jax.ShapeDtypeStruct((2 * NUM_WORKERS, NB), jnp.float32),
        scratch_types=[
            pltpu.VMEM((CHUNK,), jnp.float32),
            pltpu.VMEM((NB,), jnp.float32),
            pltpu.VMEM((NB,), jnp.float32),
        ],
        compiler_params=pltpu.CompilerParams(needs_layout_passes=False),
    )(_sc_hist_kernel)
    return fn(neg_flat)


def kernel(gt_centerness, pred_binary):
    gt2d = gt_centerness.reshape(ROWS, COLS)
    pred2d = pred_binary.reshape(ROWS, COLS)

    neg_loss, stats = _tc_bce(gt2d, pred2d)

    hist = _sc_hist(neg_loss.reshape(-1))
    cnt = jnp.sum(hist[0::2, :], axis=0)
    sm = jnp.sum(hist[1::2, :], axis=0)

    pos = jnp.sum(stats[:, 0, 0])
    ign = jnp.sum(stats[:, 0, 1])
    pos_loss_sum = jnp.sum(stats[:, 0, 2])

    neg = N_TOTAL - pos - ign
    k = jnp.floor(jnp.minimum(jnp.maximum(pos, 1.0) * NEG_RATIO, neg))

    cnt_d = cnt[::-1]
    sum_d = sm[::-1]
    ccnt = jnp.cumsum(cnt_d)
    csum = jnp.cumsum(sum_d)
    above_cnt = ccnt - cnt_d
    above_sum = csum - sum_d
    idx = jnp.argmax(ccnt >= k)
    take = k - above_cnt[idx]
    avg = sum_d[idx] / jnp.maximum(cnt_d[idx], 1.0)
    topk_sum = above_sum[idx] + take * avg

    return WEIGHT * (NEG_RATIO * pos_loss_sum + topk_sum) / (pos + k + EPS)


# speedup vs baseline: 23.4988x; 1.4338x over previous
import functools

import jax
import jax.numpy as jnp
from jax import lax
from jax.experimental import pallas as pl
from jax.experimental.pallas import tpu as pltpu
from jax.experimental.pallas import tpu_sc as plsc

NEG_RATIO = 3.0
EPS = 1e-4
WEIGHT = 1.0

N_TOTAL = 16 * 512 * 512
TC_GRID = 16
TC_BLK = N_TOTAL // TC_GRID

NUM_WORKERS = 32
PER_TILE = N_TOTAL // NUM_WORKERS
CHUNK = 16384
N_CHUNKS = PER_TILE // CHUNK
NB = 8576
SHIFT = 17


def _bce_body(gt_ref, pred_ref, neg_ref, stats_ref):
    g = gt_ref[...]
    p = pred_ref[...]
    logp = jnp.maximum(jnp.log(p), -100.0)
    log1mp = jnp.maximum(jnp.log(1.0 - p), -100.0)
    loss = -(g * logp + (1.0 - g) * log1mp)
    pos_m = g >= 0.9
    ign_m = jnp.logical_and(g >= 0.8, g < 0.9)
    neg_m = g < 0.8
    neg_ref[...] = jnp.where(neg_m, loss, 0.0)
    pos_cnt = jnp.sum(pos_m.astype(jnp.float32))
    ign_cnt = jnp.sum(ign_m.astype(jnp.float32))
    pos_sum = jnp.sum(jnp.where(pos_m, loss, 0.0))
    lane = lax.broadcasted_iota(jnp.int32, (1, 1, 128), 2)
    stats_ref[...] = jnp.where(
        lane == 0, pos_cnt,
        jnp.where(lane == 1, ign_cnt,
                  jnp.where(lane == 2, pos_sum, 0.0)))


def _tc_bce(gt1d, pred1d):
    return pl.pallas_call(
        _bce_body,
        grid=(TC_GRID,),
        in_specs=[
            pl.BlockSpec((TC_BLK,), lambda i: (i,)),
            pl.BlockSpec((TC_BLK,), lambda i: (i,)),
        ],
        out_specs=[
            pl.BlockSpec((TC_BLK,), lambda i: (i,)),
            pl.BlockSpec((1, 1, 128), lambda i: (i, 0, 0)),
        ],
        out_shape=[
            jax.ShapeDtypeStruct((N_TOTAL,), jnp.float32),
            jax.ShapeDtypeStruct((TC_GRID, 1, 128), jnp.float32),
        ],
    )(gt1d, pred1d)


def _sc_hist_kernel(neg_hbm, cnt_hbm, sum_hbm, buf, cnt_ref, sum_ref):
    c = lax.axis_index("c")
    s = lax.axis_index("s")
    wid = s * 2 + c
    base = wid * PER_TILE

    zeros16 = jnp.zeros((16,), jnp.float32)
    ones16 = jnp.ones((16,), jnp.float32)

    def zero_body(i, carry):
        cnt_ref[pl.ds(i * 16, 16)] = zeros16
        sum_ref[pl.ds(i * 16, 16)] = zeros16
        return carry

    lax.fori_loop(0, NB // 16, zero_body, 0, unroll=8)

    def chunk_body(ci, carry):
        pltpu.sync_copy(neg_hbm.at[pl.ds(base + ci * CHUNK, CHUNK)], buf)

        def inner(j, c2):
            x = buf[pl.ds(j * 16, 16)]
            bits = lax.bitcast_convert_type(x, jnp.int32)
            m = bits != 0
            b = jnp.minimum(lax.shift_right_logical(bits, SHIFT), NB - 1)
            plsc.addupdate_scatter(cnt_ref, [b], ones16, mask=m)
            plsc.addupdate_scatter(sum_ref, [b], x, mask=m)
            return c2

        lax.fori_loop(0, CHUNK // 16, inner, 0, unroll=8)
        return carry

    lax.fori_loop(0, N_CHUNKS, chunk_body, 0)

    pltpu.sync_copy(cnt_ref, cnt_hbm.at[wid])
    pltpu.sync_copy(sum_ref, sum_hbm.at[wid])


def _sc_hist(neg_flat):
    mesh = plsc.VectorSubcoreMesh(core_axis_name="c", subcore_axis_name="s")
    fn = functools.partial(
        pl.kernel,
        mesh=mesh,
        out_type=[
            jax.ShapeDtypeStruct((NUM_WORKERS, NB), jnp.float32),
            jax.ShapeDtypeStruct((NUM_WORKERS, NB), jnp.float32),
        ],
        scratch_types=[
            pltpu.VMEM((CHUNK,), jnp.float32),
            pltpu.VMEM((NB,), jnp.float32),
            pltpu.VMEM((NB,), jnp.float32),
        ],
        compiler_params=pltpu.CompilerParams(needs_layout_passes=False),
    )(_sc_hist_kernel)
    return fn(neg_flat)


def kernel(gt_centerness, pred_binary):
    gt1d = gt_centerness.reshape(-1)
    pred1d = pred_binary.reshape(-1)

    neg_loss, stats = _tc_bce(gt1d, pred1d)

    cnt_t, sum_t = _sc_hist(neg_loss)
    cnt = jnp.sum(cnt_t, axis=0)
    sm = jnp.sum(sum_t, axis=0)

    pos = jnp.sum(stats[:, 0, 0])
    ign = jnp.sum(stats[:, 0, 1])
    pos_loss_sum = jnp.sum(stats[:, 0, 2])

    neg = N_TOTAL - pos - ign
    k = jnp.floor(jnp.minimum(jnp.maximum(pos, 1.0) * NEG_RATIO, neg))

    cnt_d = cnt[::-1]
    sum_d = sm[::-1]
    ccnt = jnp.cumsum(cnt_d)
    csum = jnp.cumsum(sum_d)
    above_cnt = ccnt - cnt_d
    above_sum = csum - sum_d
    idx = jnp.argmax(ccnt >= k)
    take = k - above_cnt[idx]
    avg = sum_d[idx] / jnp.maximum(cnt_d[idx], 1.0)
    topk_sum = jnp.where(ccnt[-1] >= k,
                         above_sum[idx] + take * avg,
                         csum[-1])

    return WEIGHT * (NEG_RATIO * pos_loss_sum + topk_sum) / (pos + k + EPS)


# speedup vs baseline: 26.5406x; 1.1294x over previous
import functools

import jax
import jax.numpy as jnp
from jax import lax
from jax.experimental import pallas as pl
from jax.experimental.pallas import tpu as pltpu
from jax.experimental.pallas import tpu_sc as plsc

NEG_RATIO = 3.0
EPS = 1e-4
WEIGHT = 1.0

N_TOTAL = 16 * 512 * 512
N_SPLIT = 2
HALF = N_TOTAL // N_SPLIT
TC_GRID = 8
TC_BLK = HALF // TC_GRID

NUM_WORKERS = 32
PER_TILE = HALF // NUM_WORKERS
CHUNK = 16384
N_CHUNKS = PER_TILE // CHUNK
NB = 8576
SHIFT = 17


def _bce_body(gt_ref, pred_ref, neg_ref, stats_ref):
    g = gt_ref[...]
    p = pred_ref[...]
    logp = jnp.maximum(jnp.log(p), -100.0)
    log1mp = jnp.maximum(jnp.log(1.0 - p), -100.0)
    loss = -(g * logp + (1.0 - g) * log1mp)
    pos_m = g >= 0.9
    ign_m = jnp.logical_and(g >= 0.8, g < 0.9)
    neg_m = g < 0.8
    neg_ref[...] = jnp.where(neg_m, loss, 0.0)
    pos_cnt = jnp.sum(pos_m.astype(jnp.float32))
    ign_cnt = jnp.sum(ign_m.astype(jnp.float32))
    pos_sum = jnp.sum(jnp.where(pos_m, loss, 0.0))
    lane = lax.broadcasted_iota(jnp.int32, (1, 1, 128), 2)
    stats_ref[...] = jnp.where(
        lane == 0, pos_cnt,
        jnp.where(lane == 1, ign_cnt,
                  jnp.where(lane == 2, pos_sum, 0.0)))


def _tc_bce(gt1d, pred1d):
    return pl.pallas_call(
        _bce_body,
        grid=(TC_GRID,),
        in_specs=[
            pl.BlockSpec((TC_BLK,), lambda i: (i,)),
            pl.BlockSpec((TC_BLK,), lambda i: (i,)),
        ],
        out_specs=[
            pl.BlockSpec((TC_BLK,), lambda i: (i,)),
            pl.BlockSpec((1, 1, 128), lambda i: (i, 0, 0)),
        ],
        out_shape=[
            jax.ShapeDtypeStruct((HALF,), jnp.float32),
            jax.ShapeDtypeStruct((TC_GRID, 1, 128), jnp.float32),
        ],
    )(gt1d, pred1d)


def _sc_hist_kernel(neg_hbm, cnt_hbm, sum_hbm, buf, cnt_ref, sum_ref):
    c = lax.axis_index("c")
    s = lax.axis_index("s")
    wid = s * 2 + c
    base = wid * PER_TILE

    zeros16 = jnp.zeros((16,), jnp.float32)
    ones16 = jnp.ones((16,), jnp.float32)

    def zero_body(i, carry):
        cnt_ref[pl.ds(i * 16, 16)] = zeros16
        sum_ref[pl.ds(i * 16, 16)] = zeros16
        return carry

    lax.fori_loop(0, NB // 16, zero_body, 0, unroll=8)

    def chunk_body(ci, carry):
        pltpu.sync_copy(neg_hbm.at[pl.ds(base + ci * CHUNK, CHUNK)], buf)

        def inner(j, c2):
            x = buf[pl.ds(j * 16, 16)]
            bits = lax.bitcast_convert_type(x, jnp.int32)
            m = bits != 0
            b = jnp.minimum(lax.shift_right_logical(bits, SHIFT), NB - 1)
            plsc.addupdate_scatter(cnt_ref, [b], ones16, mask=m)
            plsc.addupdate_scatter(sum_ref, [b], x, mask=m)
            return c2

        lax.fori_loop(0, CHUNK // 16, inner, 0, unroll=8)
        return carry

    lax.fori_loop(0, N_CHUNKS, chunk_body, 0)

    pltpu.sync_copy(cnt_ref, cnt_hbm.at[wid])
    pltpu.sync_copy(sum_ref, sum_hbm.at[wid])


def _sc_hist(neg_flat):
    mesh = plsc.VectorSubcoreMesh(core_axis_name="c", subcore_axis_name="s")
    fn = functools.partial(
        pl.kernel,
        mesh=mesh,
        out_type=[
            jax.ShapeDtypeStruct((NUM_WORKERS, NB), jnp.float32),
            jax.ShapeDtypeStruct((NUM_WORKERS, NB), jnp.float32),
        ],
        scratch_types=[
            pltpu.VMEM((CHUNK,), jnp.float32),
            pltpu.VMEM((NB,), jnp.float32),
            pltpu.VMEM((NB,), jnp.float32),
        ],
        compiler_params=pltpu.CompilerParams(needs_layout_passes=False),
    )(_sc_hist_kernel)
    return fn(neg_flat)


def kernel(gt_centerness, pred_binary):
    gt1d = gt_centerness.reshape(-1)
    pred1d = pred_binary.reshape(-1)

    neg0, stats0 = _tc_bce(gt1d[:HALF], pred1d[:HALF])
    cnt0, sum0 = _sc_hist(neg0)
    neg1, stats1 = _tc_bce(gt1d[HALF:], pred1d[HALF:])
    cnt1, sum1 = _sc_hist(neg1)

    cnt = jnp.sum(cnt0, axis=0) + jnp.sum(cnt1, axis=0)
    sm = jnp.sum(sum0, axis=0) + jnp.sum(sum1, axis=0)

    stats = jnp.concatenate([stats0, stats1], axis=0)
    pos = jnp.sum(stats[:, 0, 0])
    ign = jnp.sum(stats[:, 0, 1])
    pos_loss_sum = jnp.sum(stats[:, 0, 2])

    neg = N_TOTAL - pos - ign
    k = jnp.floor(jnp.minimum(jnp.maximum(pos, 1.0) * NEG_RATIO, neg))

    cnt_d = cnt[::-1]
    sum_d = sm[::-1]
    ccnt = jnp.cumsum(cnt_d)
    csum = jnp.cumsum(sum_d)
    above_cnt = ccnt - cnt_d
    above_sum = csum - sum_d
    idx = jnp.argmax(ccnt >= k)
    take = k - above_cnt[idx]
    avg = sum_d[idx] / jnp.maximum(cnt_d[idx], 1.0)
    topk_sum = jnp.where(ccnt[-1] >= k,
                         above_sum[idx] + take * avg,
                         csum[-1])

    return WEIGHT * (NEG_RATIO * pos_loss_sum + topk_sum) / (pos + k + EPS)


# speedup vs baseline: 27.5880x; 1.0395x over previous
import functools

import jax
import jax.numpy as jnp
from jax import lax
from jax.experimental import pallas as pl
from jax.experimental.pallas import tpu as pltpu
from jax.experimental.pallas import tpu_sc as plsc

NEG_RATIO = 3.0
EPS = 1e-4
WEIGHT = 1.0

N_TOTAL = 16 * 512 * 512
N_SPLIT = 2
HALF = N_TOTAL // N_SPLIT
TC_GRID = 8
TC_BLK = HALF // TC_GRID

NUM_WORKERS = 32
PER_TILE = HALF // NUM_WORKERS
CHUNK = 16384
N_CHUNKS = PER_TILE // CHUNK
NB = 8576
SHIFT = 17


def _bce_body(gt_ref, pred_ref, neg_ref, stats_ref):
    g = gt_ref[...]
    p = pred_ref[...]
    logp = jnp.maximum(jnp.log(p), -100.0)
    log1mp = jnp.maximum(jnp.log(1.0 - p), -100.0)
    loss = -(g * logp + (1.0 - g) * log1mp)
    pos_m = g >= 0.9
    ign_m = jnp.logical_and(g >= 0.8, g < 0.9)
    neg_m = g < 0.8
    neg_ref[...] = jnp.where(neg_m, loss, 0.0)
    pos_cnt = jnp.sum(pos_m.astype(jnp.float32))
    ign_cnt = jnp.sum(ign_m.astype(jnp.float32))
    pos_sum = jnp.sum(jnp.where(pos_m, loss, 0.0))
    lane = lax.broadcasted_iota(jnp.int32, (1, 1, 128), 2)
    stats_ref[...] = jnp.where(
        lane == 0, pos_cnt,
        jnp.where(lane == 1, ign_cnt,
                  jnp.where(lane == 2, pos_sum, 0.0)))


def _tc_bce(gt1d, pred1d):
    return pl.pallas_call(
        _bce_body,
        grid=(TC_GRID,),
        in_specs=[
            pl.BlockSpec((TC_BLK,), lambda i: (i,)),
            pl.BlockSpec((TC_BLK,), lambda i: (i,)),
        ],
        out_specs=[
            pl.BlockSpec((TC_BLK,), lambda i: (i,)),
            pl.BlockSpec((1, 1, 128), lambda i: (i, 0, 0)),
        ],
        out_shape=[
            jax.ShapeDtypeStruct((HALF,), jnp.float32),
            jax.ShapeDtypeStruct((TC_GRID, 1, 128), jnp.float32),
        ],
    )(gt1d, pred1d)


def _sc_hist_kernel(neg_hbm, sum_hbm, buf, sum_ref):
    c = lax.axis_index("c")
    s = lax.axis_index("s")
    wid = s * 2 + c
    base = wid * PER_TILE

    zeros16 = jnp.zeros((16,), jnp.float32)

    def zero_body(i, carry):
        sum_ref[pl.ds(i * 16, 16)] = zeros16
        return carry

    lax.fori_loop(0, NB // 16, zero_body, 0, unroll=8)

    def chunk_body(ci, carry):
        pltpu.sync_copy(neg_hbm.at[pl.ds(base + ci * CHUNK, CHUNK)], buf)

        def inner(j, c2):
            x = buf[pl.ds(j * 16, 16)]
            bits = lax.bitcast_convert_type(x, jnp.int32)
            m = bits != 0
            b = jnp.minimum(lax.shift_right_logical(bits, SHIFT), NB - 1)
            plsc.addupdate_scatter(sum_ref, [b], x, mask=m)
            return c2

        lax.fori_loop(0, CHUNK // 16, inner, 0, unroll=8)
        return carry

    lax.fori_loop(0, N_CHUNKS, chunk_body, 0)

    pltpu.sync_copy(sum_ref, sum_hbm.at[wid])


def _sc_hist(neg_flat):
    mesh = plsc.VectorSubcoreMesh(core_axis_name="c", subcore_axis_name="s")
    fn = functools.partial(
        pl.kernel,
        mesh=mesh,
        out_type=jax.ShapeDtypeStruct((NUM_WORKERS, NB), jnp.float32),
        scratch_types=[
            pltpu.VMEM((CHUNK,), jnp.float32),
            pltpu.VMEM((NB,), jnp.float32),
        ],
        compiler_params=pltpu.CompilerParams(needs_layout_passes=False),
    )(_sc_hist_kernel)
    return fn(neg_flat)


def kernel(gt_centerness, pred_binary):
    gt1d = gt_centerness.reshape(-1)
    pred1d = pred_binary.reshape(-1)

    neg0, stats0 = _tc_bce(gt1d[:HALF], pred1d[:HALF])
    sum0 = _sc_hist(neg0)
    neg1, stats1 = _tc_bce(gt1d[HALF:], pred1d[HALF:])
    sum1 = _sc_hist(neg1)

    sm = jnp.sum(sum0, axis=0) + jnp.sum(sum1, axis=0)
    centers = lax.bitcast_convert_type(
        (jnp.arange(NB, dtype=jnp.int32) * 2 + 1) << (SHIFT - 1),
        jnp.float32)
    cnt = jnp.where(centers >= 1e-30, jnp.round(sm / centers), 0.0)

    stats = jnp.concatenate([stats0, stats1], axis=0)
    pos = jnp.sum(stats[:, 0, 0])
    ign = jnp.sum(stats[:, 0, 1])
    pos_loss_sum = jnp.sum(stats[:, 0, 2])

    neg = N_TOTAL - pos - ign
    k = jnp.floor(jnp.minimum(jnp.maximum(pos, 1.0) * NEG_RATIO, neg))

    cnt_d = cnt[::-1]
    sum_d = sm[::-1]
    ccnt = jnp.cumsum(cnt_d)
    csum = jnp.cumsum(sum_d)
    above_cnt = ccnt - cnt_d
    above_sum = csum - sum_d
    idx = jnp.argmax(ccnt >= k)
    take = k - above_cnt[idx]
    avg = sum_d[idx] / jnp.maximum(cnt_d[idx], 1.0)
    topk_sum = jnp.where(ccnt[-1] >= k,
                         above_sum[idx] + take * avg,
                         csum[-1])

    return WEIGHT * (NEG_RATIO * pos_loss_sum + topk_sum) / (pos + k + EPS)


# speedup vs baseline: 29.8380x; 1.0816x over previous
import functools

import jax
import jax.numpy as jnp
from jax import lax
from jax.experimental import pallas as pl
from jax.experimental.pallas import tpu as pltpu
from jax.experimental.pallas import tpu_sc as plsc

NEG_RATIO = 3.0
EPS = 1e-4
WEIGHT = 1.0

N_TOTAL = 16 * 512 * 512
N_SPLIT = 2
HALF = N_TOTAL // N_SPLIT
TC_GRID = 8
TC_ROWS = HALF // 128 // TC_GRID

NUM_WORKERS = 32
PER_TILE = HALF // NUM_WORKERS
CHUNK = 16384
N_CHUNKS = PER_TILE // CHUNK
NB = 8576
SHIFT = 17


def _bce_body(gt_ref, pred_ref, neg_ref, stats_ref):
    g = gt_ref[...]
    p = pred_ref[...]
    logp = jnp.maximum(jnp.log(p), -100.0)
    log1mp = jnp.maximum(jnp.log(1.0 - p), -100.0)
    loss = -(g * logp + (1.0 - g) * log1mp)
    pos_m = g >= 0.9
    ign_m = jnp.logical_and(g >= 0.8, g < 0.9)
    neg_ref[...] = jnp.where(g < 0.8, loss, 0.0)
    pos_v = jnp.sum(pos_m.astype(jnp.float32), axis=0)
    ign_v = jnp.sum(ign_m.astype(jnp.float32), axis=0)
    psum_v = jnp.sum(jnp.where(pos_m, loss, 0.0), axis=0)
    stats_ref[...] = jnp.stack([pos_v, ign_v, psum_v], axis=0)[None]


def _tc_bce(gt2d, pred2d):
    return pl.pallas_call(
        _bce_body,
        grid=(TC_GRID,),
        in_specs=[
            pl.BlockSpec((TC_ROWS, 128), lambda i: (i, 0)),
            pl.BlockSpec((TC_ROWS, 128), lambda i: (i, 0)),
        ],
        out_specs=[
            pl.BlockSpec((TC_ROWS, 128), lambda i: (i, 0)),
            pl.BlockSpec((1, 3, 128), lambda i: (i, 0, 0)),
        ],
        out_shape=[
            jax.ShapeDtypeStruct((TC_ROWS * TC_GRID, 128), jnp.float32),
            jax.ShapeDtypeStruct((TC_GRID, 3, 128), jnp.float32),
        ],
    )(gt2d, pred2d)


def _sc_hist_kernel(neg_hbm, sum_hbm, buf, sum_ref):
    c = lax.axis_index("c")
    s = lax.axis_index("s")
    wid = s * 2 + c
    base = wid * PER_TILE

    zeros16 = jnp.zeros((16,), jnp.float32)

    def zero_body(i, carry):
        sum_ref[pl.ds(i * 16, 16)] = zeros16
        return carry

    lax.fori_loop(0, NB // 16, zero_body, 0, unroll=8)

    def chunk_body(ci, carry):
        pltpu.sync_copy(neg_hbm.at[pl.ds(base + ci * CHUNK, CHUNK)], buf)

        def inner(j, c2):
            x = buf[pl.ds(j * 16, 16)]
            bits = lax.bitcast_convert_type(x, jnp.int32)
            b = lax.shift_right_logical(bits, SHIFT)
            plsc.addupdate_scatter(sum_ref, [b], x)
            return c2

        lax.fori_loop(0, CHUNK // 16, inner, 0, unroll=8)
        return carry

    lax.fori_loop(0, N_CHUNKS, chunk_body, 0)

    pltpu.sync_copy(sum_ref, sum_hbm.at[wid])


def _sc_hist(neg_flat):
    mesh = plsc.VectorSubcoreMesh(core_axis_name="c", subcore_axis_name="s")
    fn = functools.partial(
        pl.kernel,
        mesh=mesh,
        out_type=jax.ShapeDtypeStruct((NUM_WORKERS, NB), jnp.float32),
        scratch_types=[
            pltpu.VMEM((CHUNK,), jnp.float32),
            pltpu.VMEM((NB,), jnp.float32),
        ],
        compiler_params=pltpu.CompilerParams(needs_layout_passes=False),
    )(_sc_hist_kernel)
    return fn(neg_flat)


def kernel(gt_centerness, pred_binary):
    gt2d = gt_centerness.reshape(-1, 128)
    pred2d = pred_binary.reshape(-1, 128)
    hrows = HALF // 128

    neg0, stats0 = _tc_bce(gt2d[:hrows], pred2d[:hrows])
    sum0 = _sc_hist(neg0.reshape(-1))
    neg1, stats1 = _tc_bce(gt2d[hrows:], pred2d[hrows:])
    sum1 = _sc_hist(neg1.reshape(-1))

    sm = jnp.sum(sum0, axis=0) + jnp.sum(sum1, axis=0)
    centers = lax.bitcast_convert_type(
        (jnp.arange(NB, dtype=jnp.int32) * 2 + 1) << (SHIFT - 1),
        jnp.float32)
    cnt = jnp.where(centers >= 1e-30, jnp.round(sm / centers), 0.0)

    stats = jnp.concatenate([stats0, stats1], axis=0)
    pos = jnp.sum(stats[:, 0, :])
    ign = jnp.sum(stats[:, 1, :])
    pos_loss_sum = jnp.sum(stats[:, 2, :])

    neg = N_TOTAL - pos - ign
    k = jnp.floor(jnp.minimum(jnp.maximum(pos, 1.0) * NEG_RATIO, neg))

    cnt_d = cnt[::-1]
    sum_d = sm[::-1]
    ccnt = jnp.cumsum(cnt_d)
    csum = jnp.cumsum(sum_d)
    above_cnt = ccnt - cnt_d
    above_sum = csum - sum_d
    idx = jnp.argmax(ccnt >= k)
    take = k - above_cnt[idx]
    avg = sum_d[idx] / jnp.maximum(cnt_d[idx], 1.0)
    topk_sum = jnp.where(ccnt[-1] >= k,
                         above_sum[idx] + take * avg,
                         csum[-1])

    return WEIGHT * (NEG_RATIO * pos_loss_sum + topk_sum) / (pos + k + EPS)


# speedup vs baseline: 30.0068x; 1.0057x over previous
import functools

import jax
import jax.numpy as jnp
from jax import lax
from jax.experimental import pallas as pl
from jax.experimental.pallas import tpu as pltpu
from jax.experimental.pallas import tpu_sc as plsc

NEG_RATIO = 3.0
EPS = 1e-4
WEIGHT = 1.0

N_TOTAL = 16 * 512 * 512
N_SPLIT = 2
HALF = N_TOTAL // N_SPLIT
TC_GRID = 8
TC_ROWS = HALF // 128 // TC_GRID

NUM_WORKERS = 32
PER_TILE = HALF // NUM_WORKERS
CHUNK = 16384
N_CHUNKS = PER_TILE // CHUNK
NB = 8576
SHIFT = 17


def _bce_body(gt_ref, pred_ref, neg_ref, stats_ref):
    g = gt_ref[...].reshape(TC_ROWS, 128)
    p = pred_ref[...].reshape(TC_ROWS, 128)
    logp = jnp.maximum(jnp.log(p), -100.0)
    log1mp = jnp.maximum(jnp.log(1.0 - p), -100.0)
    loss = -(g * logp + (1.0 - g) * log1mp)
    pos_m = g >= 0.9
    ign_m = jnp.logical_and(g >= 0.8, g < 0.9)
    neg_ref[...] = jnp.where(g < 0.8, loss, 0.0).reshape(-1)
    pos_v = jnp.sum(pos_m.astype(jnp.float32), axis=0)
    ign_v = jnp.sum(ign_m.astype(jnp.float32), axis=0)
    psum_v = jnp.sum(jnp.where(pos_m, loss, 0.0), axis=0)
    stats_ref[...] = jnp.stack([pos_v, ign_v, psum_v], axis=0)[None]


TC_BLK = TC_ROWS * 128


def _tc_bce(gt1d, pred1d):
    return pl.pallas_call(
        _bce_body,
        grid=(TC_GRID,),
        in_specs=[
            pl.BlockSpec((TC_BLK,), lambda i: (i,)),
            pl.BlockSpec((TC_BLK,), lambda i: (i,)),
        ],
        out_specs=[
            pl.BlockSpec((TC_BLK,), lambda i: (i,)),
            pl.BlockSpec((1, 3, 128), lambda i: (i, 0, 0)),
        ],
        out_shape=[
            jax.ShapeDtypeStruct((HALF,), jnp.float32),
            jax.ShapeDtypeStruct((TC_GRID, 3, 128), jnp.float32),
        ],
    )(gt1d, pred1d)


def _sc_hist_kernel(neg_hbm, sum_hbm, buf, sum_ref):
    c = lax.axis_index("c")
    s = lax.axis_index("s")
    wid = s * 2 + c
    base = wid * PER_TILE

    zeros16 = jnp.zeros((16,), jnp.float32)

    def zero_body(i, carry):
        sum_ref[pl.ds(i * 16, 16)] = zeros16
        return carry

    lax.fori_loop(0, NB // 16, zero_body, 0, unroll=8)

    def chunk_body(ci, carry):
        pltpu.sync_copy(neg_hbm.at[pl.ds(base + ci * CHUNK, CHUNK)], buf)

        def inner(j, c2):
            x = buf[pl.ds(j * 16, 16)]
            bits = lax.bitcast_convert_type(x, jnp.int32)
            b = lax.shift_right_logical(bits, SHIFT)
            plsc.addupdate_scatter(sum_ref, [b], x)
            return c2

        lax.fori_loop(0, CHUNK // 16, inner, 0, unroll=8)
        return carry

    lax.fori_loop(0, N_CHUNKS, chunk_body, 0)

    pltpu.sync_copy(sum_ref, sum_hbm.at[wid])


def _sc_hist(neg_flat):
    mesh = plsc.VectorSubcoreMesh(core_axis_name="c", subcore_axis_name="s")
    fn = functools.partial(
        pl.kernel,
        mesh=mesh,
        out_type=jax.ShapeDtypeStruct((NUM_WORKERS, NB), jnp.float32),
        scratch_types=[
            pltpu.VMEM((CHUNK,), jnp.float32),
            pltpu.VMEM((NB,), jnp.float32),
        ],
        compiler_params=pltpu.CompilerParams(needs_layout_passes=False),
    )(_sc_hist_kernel)
    return fn(neg_flat)


def kernel(gt_centerness, pred_binary):
    gt1d = gt_centerness.reshape(-1)
    pred1d = pred_binary.reshape(-1)

    neg0, stats0 = _tc_bce(gt1d[:HALF], pred1d[:HALF])
    sum0 = _sc_hist(neg0)
    neg1, stats1 = _tc_bce(gt1d[HALF:], pred1d[HALF:])
    sum1 = _sc_hist(neg1)

    sm = jnp.sum(sum0, axis=0) + jnp.sum(sum1, axis=0)
    centers = lax.bitcast_convert_type(
        (jnp.arange(NB, dtype=jnp.int32) * 2 + 1) << (SHIFT - 1),
        jnp.float32)
    cnt = jnp.where(centers >= 1e-30, jnp.round(sm / centers), 0.0)

    stats = jnp.concatenate([stats0, stats1], axis=0)
    pos = jnp.sum(stats[:, 0, :])
    ign = jnp.sum(stats[:, 1, :])
    pos_loss_sum = jnp.sum(stats[:, 2, :])

    neg = N_TOTAL - pos - ign
    k = jnp.floor(jnp.minimum(jnp.maximum(pos, 1.0) * NEG_RATIO, neg))

    cnt_d = cnt[::-1]
    sum_d = sm[::-1]
    ccnt = jnp.cumsum(cnt_d)
    csum = jnp.cumsum(sum_d)
    above_cnt = ccnt - cnt_d
    above_sum = csum - sum_d
    idx = jnp.argmax(ccnt >= k)
    take = k - above_cnt[idx]
    avg = sum_d[idx] / jnp.maximum(cnt_d[idx], 1.0)
    topk_sum = jnp.where(ccnt[-1] >= k,
                         above_sum[idx] + take * avg,
                         csum[-1])

    return WEIGHT * (NEG_RATIO * pos_loss_sum + topk_sum) / (pos + k + EPS)


# speedup vs baseline: 31.2769x; 1.0423x over previous
import functools

import jax
import jax.numpy as jnp
from jax import lax
from jax.experimental import pallas as pl
from jax.experimental.pallas import tpu as pltpu
from jax.experimental.pallas import tpu_sc as plsc

NEG_RATIO = 3.0
EPS = 1e-4
WEIGHT = 1.0

N_TOTAL = 16 * 512 * 512
N_SPLIT = 2
HALF = N_TOTAL // N_SPLIT
TC_GRID = 8
TC_ROWS = HALF // 128 // TC_GRID

NUM_WORKERS = 32
PER_TILE = HALF // NUM_WORKERS
CHUNK = 16384
N_CHUNKS = PER_TILE // CHUNK
NB = 8576
SHIFT = 17


def _bce_body(gt_ref, pred_ref, neg_ref, stats_ref):
    g = gt_ref[...].reshape(TC_ROWS, 128)
    p = pred_ref[...].reshape(TC_ROWS, 128)
    logp = jnp.maximum(jnp.log(p), -100.0)
    log1mp = jnp.maximum(jnp.log(1.0 - p), -100.0)
    loss = -(g * logp + (1.0 - g) * log1mp)
    pos_m = g >= 0.9
    ign_m = jnp.logical_and(g >= 0.8, g < 0.9)
    neg_ref[...] = jnp.where(g < 0.8, loss, 0.0).reshape(-1)
    pos_v = jnp.sum(pos_m.astype(jnp.float32), axis=0)
    ign_v = jnp.sum(ign_m.astype(jnp.float32), axis=0)
    psum_v = jnp.sum(jnp.where(pos_m, loss, 0.0), axis=0)
    stats_ref[...] = jnp.stack([pos_v, ign_v, psum_v], axis=0)[None]


TC_BLK = TC_ROWS * 128


def _tc_bce(gt1d, pred1d):
    return pl.pallas_call(
        _bce_body,
        grid=(TC_GRID,),
        in_specs=[
            pl.BlockSpec((TC_BLK,), lambda i: (i,)),
            pl.BlockSpec((TC_BLK,), lambda i: (i,)),
        ],
        out_specs=[
            pl.BlockSpec((TC_BLK,), lambda i: (i,)),
            pl.BlockSpec((1, 3, 128), lambda i: (i, 0, 0)),
        ],
        out_shape=[
            jax.ShapeDtypeStruct((HALF,), jnp.float32),
            jax.ShapeDtypeStruct((TC_GRID, 3, 128), jnp.float32),
        ],
    )(gt1d, pred1d)


def _sc_hist_kernel(neg_hbm, sum_hbm, buf, sum_ref):
    c = lax.axis_index("c")
    s = lax.axis_index("s")
    wid = s * 2 + c
    base = wid * PER_TILE

    zeros16 = jnp.zeros((16,), jnp.float32)

    def zero_body(i, carry):
        sum_ref[pl.ds(i * 16, 16)] = zeros16
        return carry

    lax.fori_loop(0, NB // 16, zero_body, 0, unroll=8)

    def chunk_body(ci, carry):
        pltpu.sync_copy(neg_hbm.at[pl.ds(base + ci * CHUNK, CHUNK)], buf)

        def inner(j, c2):
            x = buf[pl.ds(j * 16, 16)]
            bits = lax.bitcast_convert_type(x, jnp.int32)
            m = bits != 0
            b = lax.shift_right_logical(bits, SHIFT)
            plsc.addupdate_scatter(sum_ref, [b], x, mask=m)
            return c2

        lax.fori_loop(0, CHUNK // 16, inner, 0, unroll=8)
        return carry

    lax.fori_loop(0, N_CHUNKS, chunk_body, 0)

    pltpu.sync_copy(sum_ref, sum_hbm.at[wid])


def _sc_hist(neg_flat):
    mesh = plsc.VectorSubcoreMesh(core_axis_name="c", subcore_axis_name="s")
    fn = functools.partial(
        pl.kernel,
        mesh=mesh,
        out_type=jax.ShapeDtypeStruct((NUM_WORKERS, NB), jnp.float32),
        scratch_types=[
            pltpu.VMEM((CHUNK,), jnp.float32),
            pltpu.VMEM((NB,), jnp.float32),
        ],
        compiler_params=pltpu.CompilerParams(needs_layout_passes=False),
    )(_sc_hist_kernel)
    return fn(neg_flat)


def kernel(gt_centerness, pred_binary):
    gt1d = gt_centerness.reshape(-1)
    pred1d = pred_binary.reshape(-1)

    neg0, stats0 = _tc_bce(gt1d[:HALF], pred1d[:HALF])
    sum0 = _sc_hist(neg0)
    neg1, stats1 = _tc_bce(gt1d[HALF:], pred1d[HALF:])
    sum1 = _sc_hist(neg1)

    sm = jnp.sum(sum0, axis=0) + jnp.sum(sum1, axis=0)
    centers = lax.bitcast_convert_type(
        (jnp.arange(NB, dtype=jnp.int32) * 2 + 1) << (SHIFT - 1),
        jnp.float32)
    cnt = jnp.where(centers >= 1e-30, jnp.round(sm / centers), 0.0)

    stats = jnp.concatenate([stats0, stats1], axis=0)
    pos = jnp.sum(stats[:, 0, :])
    ign = jnp.sum(stats[:, 1, :])
    pos_loss_sum = jnp.sum(stats[:, 2, :])

    neg = N_TOTAL - pos - ign
    k = jnp.floor(jnp.minimum(jnp.maximum(pos, 1.0) * NEG_RATIO, neg))

    cnt_d = cnt[::-1]
    sum_d = sm[::-1]
    ccnt = jnp.cumsum(cnt_d)
    csum = jnp.cumsum(sum_d)
    above_cnt = ccnt - cnt_d
    above_sum = csum - sum_d
    idx = jnp.argmax(ccnt >= k)
    take = k - above_cnt[idx]
    avg = sum_d[idx] / jnp.maximum(cnt_d[idx], 1.0)
    topk_sum = jnp.where(ccnt[-1] >= k,
                         above_sum[idx] + take * avg,
                         csum[-1])

    return WEIGHT * (NEG_RATIO * pos_loss_sum + topk_sum) / (pos + k + EPS)
